# baseline (speedup 1.0000x reference)
import functools

import jax
import jax.numpy as jnp
from jax import lax
from jax.experimental import pallas as pl
from jax.experimental.pallas import tpu as pltpu
from jax.experimental.pallas import tpu_sc as plsc

_EMBED = 16
_BATCH = 16384
_N_ROWS = 1000000
_REGS = 0.0001
_NC, _NS, _L = 2, 16, 16
_NW = _NC * _NS
_BPW = _BATCH // _NW
_CH = 128
_NCH = _BPW // _CH
_WIDE = 128
_RPW = _WIDE // _EMBED
_TCOLS = -(-_N_ROWS // _WIDE)
_OUT_ROWS = _TCOLS * _L

_mesh = plsc.VectorSubcoreMesh(core_axis_name="c", subcore_axis_name="s")


_TPT = 248
_UNIT = 2
_NUNIT = _TPT // _UNIT
_NBUF = 4


@functools.partial(
    pl.kernel,
    out_type=(
        jax.ShapeDtypeStruct((_OUT_ROWS, _WIDE), jnp.float32),
        jax.ShapeDtypeStruct((_OUT_ROWS, _WIDE), jnp.float32),
    ),
    mesh=_mesh,
    compiler_params=pltpu.CompilerParams(
        needs_layout_passes=False, use_tc_tiling_on_sc=True,
        disable_bounds_checks=True),
    scratch_types=(
        pltpu.VMEM((_EMBED, _UNIT * _WIDE), jnp.float32),
        pltpu.VMEM((_EMBED, _UNIT * _WIDE), jnp.float32),
        pltpu.VMEM((_EMBED, _UNIT * _WIDE), jnp.float32),
        pltpu.VMEM((_EMBED, _UNIT * _WIDE), jnp.float32),
        pltpu.VMEM((_UNIT * _EMBED, _WIDE), jnp.float32),
        pltpu.VMEM((_UNIT * _EMBED, _WIDE), jnp.float32),
        pltpu.VMEM((_UNIT * _EMBED, _WIDE), jnp.float32),
        pltpu.VMEM((_UNIT * _EMBED, _WIDE), jnp.float32),
        pltpu.SemaphoreType.DMA,
        pltpu.SemaphoreType.DMA,
        pltpu.SemaphoreType.DMA,
        pltpu.SemaphoreType.DMA,
        pltpu.SemaphoreType.DMA,
        pltpu.SemaphoreType.DMA,
        pltpu.SemaphoreType.DMA,
        pltpu.SemaphoreType.DMA,
        pltpu.VMEM((_EMBED, _L), jnp.int32),
    ),
)
def _sc_convert(uemb_t, iemb_t, urm, irm,
                i0, i1, i2, i3, o0, o1, o2, o3,
                si0, si1, si2, si3, so0, so1, so2, so3, coltab):
    wid = lax.axis_index("s") * _NC + lax.axis_index("c")
    t0 = jnp.minimum(wid * _TPT, _TCOLS - _TPT)
    iota = jnp.arange(_L, dtype=jnp.int32)
    inb = [i0, i1, i2, i3]
    outb = [o0, o1, o2, o3]
    sin = [si0, si1, si2, si3]
    sout = [so0, so1, so2, so3]
    rowv = [2 * m + lax.shift_right_logical(iota, 3) for m in range(_RPW)]
    colbase = jnp.left_shift(jnp.bitwise_and(iota, 7), 4)
    for l in range(_EMBED):
        coltab[l, :] = colbase + jnp.bitwise_and(l + iota, 15)

    def convert_table(src, dst):
        def fetch(k, q):
            col = pl.multiple_of((t0 + k * _UNIT) * _WIDE, _WIDE)
            pltpu.async_copy(
                src.at[:, pl.ds(col, _UNIT * _WIDE)], inb[q], sin[q])

        def wait_fetch(q):
            pltpu.make_async_copy(
                src.at[:, pl.ds(0, _UNIT * _WIDE)], inb[q], sin[q]).wait()

        def flush(k, q):
            row = (t0 + k * _UNIT) * _L
            pltpu.async_copy(
                outb[q], dst.at[pl.ds(row, _UNIT * _L)], sout[q])

        def wait_flush(q):
            pltpu.make_async_copy(
                outb[q], dst.at[pl.ds(0, _UNIT * _L)], sout[q]).wait()

        def process(q):
            for u in range(_UNIT):
                for m in range(_RPW):
                    rv = rowv[m] + u * _L
                    for l in range(_EMBED):
                        cv = coltab[l, :]
                        v = inb[q][l, pl.ds(u * _WIDE + m * _L, _L)]
                        plsc.store_scatter(outb[q], [rv, cv], v)

        for q in range(_NBUF):
            fetch(jnp.int32(q), q)

        def body(p, _):
            for q in range(_NBUF):
                k = p * _NBUF + q
                wait_fetch(q)

                @pl.when(p >= 1)
                def _(q=q):
                    wait_flush(q)

                process(q)
                flush(k, q)

                @pl.when(k + _NBUF < _NUNIT)
                def _(k=k, q=q):
                    fetch(k + _NBUF, q)

            return 0

        lax.fori_loop(0, _NUNIT // _NBUF, body, 0)
        for q in range(_NBUF):
            wait_flush(q)

    convert_table(uemb_t, urm)
    convert_table(iemb_t, irm)


@functools.partial(
    pl.kernel,
    out_type=(
        jax.ShapeDtypeStruct((_BATCH,), jnp.float32),
        jax.ShapeDtypeStruct((_NW * _L,), jnp.float32),
    ),
    mesh=_mesh,
    compiler_params=pltpu.CompilerParams(
        needs_layout_passes=False, use_tc_tiling_on_sc=True,
        disable_bounds_checks=True),
    scratch_types=(
        pltpu.VMEM((_BPW,), jnp.int32),
        pltpu.VMEM((_BPW,), jnp.int32),
        pltpu.VMEM((_BPW,), jnp.int32),
        pltpu.VMEM((_BPW,), jnp.int32),
        pltpu.VMEM((_BPW,), jnp.int32),
        pltpu.VMEM((_BPW,), jnp.int32),
        pltpu.VMEM((_CH, _WIDE), jnp.float32),
        pltpu.VMEM((_CH, _WIDE), jnp.float32),
        pltpu.VMEM((_CH, _WIDE), jnp.float32),
        pltpu.VMEM((_BPW,), jnp.float32),
        pltpu.VMEM((_L,), jnp.float32),
        pltpu.SemaphoreType.DMA,
    ),
)
def _sc_gather_score(user, pos, neg, uemb, iemb, diff_out, acc_out,
                     uidx, pidx, nidx, urid, prid, nrid,
                     urows, prows, nrows, diffv, accv, sem):
    wid = lax.axis_index("s") * _NC + lax.axis_index("c")
    base = wid * _BPW
    pltpu.sync_copy(user.at[pl.ds(base, _BPW)], uidx)
    pltpu.sync_copy(pos.at[pl.ds(base, _BPW)], pidx)
    pltpu.sync_copy(neg.at[pl.ds(base, _BPW)], nidx)

    def rid_body(k, _):
        s = pl.ds(k * _L, _L)
        urid[s] = lax.shift_right_logical(uidx[s], 3)
        prid[s] = lax.shift_right_logical(pidx[s], 3)
        nrid[s] = lax.shift_right_logical(nidx[s], 3)
        return 0

    lax.fori_loop(0, _BPW // _L, rid_body, 0)

    acc = jnp.zeros((_L,), jnp.float32)
    for c in range(_NCH):
        cs = pl.ds(c * _CH, _CH)
        cp_u = pltpu.async_copy(uemb.at[urid.at[cs]], urows, sem)
        cp_p = pltpu.async_copy(iemb.at[prid.at[cs]], prows, sem)
        cp_n = pltpu.async_copy(iemb.at[nrid.at[cs]], nrows, sem)
        cp_u.wait()
        cp_p.wait()
        cp_n.wait()

        def group_body(g, acc, _c=c):
            rows = g * _L + jnp.arange(_L, dtype=jnp.int32)
            gs = pl.dslice(_c * _CH + g * _L, _L)
            iu = uidx[gs]
            ip = pidx[gs]
            inn = nidx[gs]
            cu = jnp.left_shift(jnp.bitwise_and(iu, _RPW - 1), 4)
            cp = jnp.left_shift(jnp.bitwise_and(ip, _RPW - 1), 4)
            cn = jnp.left_shift(jnp.bitwise_and(inn, _RPW - 1), 4)
            score = jnp.zeros((_L,), jnp.float32)
            for l in range(_EMBED):
                uc = plsc.load_gather(
                    urows, [rows, cu + jnp.bitwise_and(iu + l, 15)])
                pc = plsc.load_gather(
                    prows, [rows, cp + jnp.bitwise_and(ip + l, 15)])
                nc = plsc.load_gather(
                    nrows, [rows, cn + jnp.bitwise_and(inn + l, 15)])
                score = score + uc * (pc - nc)
                acc = acc + uc * uc + pc * pc + nc * nc
            diffv[pl.dslice(_c * _CH + g * _L, _L)] = score
            return acc

        acc = lax.fori_loop(0, _CH // _L, group_body, acc)

    accv[...] = acc
    pltpu.sync_copy(diffv, diff_out.at[pl.ds(base, _BPW)])
    pltpu.sync_copy(accv, acc_out.at[pl.ds(wid * _L, _L)])


def _tc_finish_body(diff_ref, acc_ref, loss_ref, reg_ref):
    d = diff_ref[...]
    ls = jnp.minimum(d, 0.0) - jnp.log1p(jnp.exp(-jnp.abs(d)))
    loss_ref[0, 0] = -jnp.sum(ls) * (1.0 / _BATCH)
    reg_ref[0, 0] = (_REGS * 0.5 / _BATCH) * jnp.sum(acc_ref[...])


def _tc_finish(diff, acc):
    loss, reg = pl.pallas_call(
        _tc_finish_body,
        out_shape=(
            jax.ShapeDtypeStruct((1, 1), jnp.float32),
            jax.ShapeDtypeStruct((1, 1), jnp.float32),
        ),
        out_specs=(
            pl.BlockSpec(memory_space=pltpu.SMEM),
            pl.BlockSpec(memory_space=pltpu.SMEM),
        ),
    )(diff.reshape(_BATCH // 128, 128), acc.reshape(_NW * _L // 128, 128))
    return loss[0, 0], reg[0, 0]


def kernel(user, pos, neg, user_embedding, item_embedding):
    urm, irm = _sc_convert(user_embedding.T, item_embedding.T)
    diff, acc = _sc_gather_score(user, pos, neg, urm, irm)
    loss, reg_loss = _tc_finish(diff, acc)
    return (loss, reg_loss)


# speedup vs baseline: 1.0038x; 1.0038x over previous
import functools

import jax
import jax.numpy as jnp
from jax import lax
from jax.experimental import pallas as pl
from jax.experimental.pallas import tpu as pltpu
from jax.experimental.pallas import tpu_sc as plsc

_EMBED = 16
_BATCH = 16384
_N_ROWS = 1000000
_REGS = 0.0001
_NC, _NS, _L = 2, 16, 16
_NW = _NC * _NS
_BPW = _BATCH // _NW
_CH = 128
_NCH = _BPW // _CH
_WIDE = 128
_RPW = _WIDE // _EMBED
_TCOLS = -(-_N_ROWS // _WIDE)
_OUT_ROWS = _TCOLS * _L

_mesh = plsc.VectorSubcoreMesh(core_axis_name="c", subcore_axis_name="s")


_TPT = 248
_UNIT = 1
_NUNIT = _TPT // _UNIT
_NBUF = 4


@functools.partial(
    pl.kernel,
    out_type=(
        jax.ShapeDtypeStruct((_OUT_ROWS, _WIDE), jnp.float32),
        jax.ShapeDtypeStruct((_OUT_ROWS, _WIDE), jnp.float32),
    ),
    mesh=_mesh,
    compiler_params=pltpu.CompilerParams(
        needs_layout_passes=False, use_tc_tiling_on_sc=True,
        disable_bounds_checks=True),
    scratch_types=(
        pltpu.VMEM((_EMBED, _UNIT * _WIDE), jnp.float32),
        pltpu.VMEM((_EMBED, _UNIT * _WIDE), jnp.float32),
        pltpu.VMEM((_EMBED, _UNIT * _WIDE), jnp.float32),
        pltpu.VMEM((_EMBED, _UNIT * _WIDE), jnp.float32),
        pltpu.VMEM((_UNIT * _EMBED, _WIDE), jnp.float32),
        pltpu.VMEM((_UNIT * _EMBED, _WIDE), jnp.float32),
        pltpu.VMEM((_UNIT * _EMBED, _WIDE), jnp.float32),
        pltpu.VMEM((_UNIT * _EMBED, _WIDE), jnp.float32),
        pltpu.SemaphoreType.DMA,
        pltpu.SemaphoreType.DMA,
        pltpu.SemaphoreType.DMA,
        pltpu.SemaphoreType.DMA,
        pltpu.SemaphoreType.DMA,
        pltpu.SemaphoreType.DMA,
        pltpu.SemaphoreType.DMA,
        pltpu.SemaphoreType.DMA,
        pltpu.VMEM((_EMBED, _L), jnp.int32),
    ),
)
def _sc_convert(uemb_t, iemb_t, urm, irm,
                i0, i1, i2, i3, o0, o1, o2, o3,
                si0, si1, si2, si3, so0, so1, so2, so3, coltab):
    wid = lax.axis_index("s") * _NC + lax.axis_index("c")
    t0 = jnp.minimum(wid * _TPT, _TCOLS - _TPT)
    iota = jnp.arange(_L, dtype=jnp.int32)
    inb = [i0, i1, i2, i3]
    outb = [o0, o1, o2, o3]
    sin = [si0, si1, si2, si3]
    sout = [so0, so1, so2, so3]
    rowv = [2 * m + lax.shift_right_logical(iota, 3) for m in range(_RPW)]
    colbase = jnp.left_shift(jnp.bitwise_and(iota, 7), 4)
    for l in range(_EMBED):
        coltab[l, :] = colbase + l

    def convert_table(src, dst):
        def fetch(k, q):
            col = pl.multiple_of((t0 + k * _UNIT) * _WIDE, _WIDE)
            pltpu.async_copy(
                src.at[:, pl.ds(col, _UNIT * _WIDE)], inb[q], sin[q])

        def wait_fetch(q):
            pltpu.make_async_copy(
                src.at[:, pl.ds(0, _UNIT * _WIDE)], inb[q], sin[q]).wait()

        def flush(k, q):
            row = (t0 + k * _UNIT) * _L
            pltpu.async_copy(
                outb[q], dst.at[pl.ds(row, _UNIT * _L)], sout[q])

        def wait_flush(q):
            pltpu.make_async_copy(
                outb[q], dst.at[pl.ds(0, _UNIT * _L)], sout[q]).wait()

        def process(q):
            for u in range(_UNIT):
                for m in range(_RPW):
                    rv = rowv[m] + u * _L
                    for l in range(_EMBED):
                        cv = coltab[l, :]
                        v = inb[q][l, pl.ds(u * _WIDE + m * _L, _L)]
                        plsc.store_scatter(outb[q], [rv, cv], v)

        for q in range(_NBUF):
            fetch(jnp.int32(q), q)

        def body(p, _):
            for q in range(_NBUF):
                k = p * _NBUF + q
                wait_fetch(q)

                @pl.when(p >= 1)
                def _(q=q):
                    wait_flush(q)

                process(q)
                flush(k, q)

                @pl.when(k + _NBUF < _NUNIT)
                def _(k=k, q=q):
                    fetch(k + _NBUF, q)

            return 0

        lax.fori_loop(0, _NUNIT // _NBUF, body, 0)
        for q in range(_NBUF):
            wait_flush(q)

    convert_table(uemb_t, urm)
    convert_table(iemb_t, irm)


@functools.partial(
    pl.kernel,
    out_type=(
        jax.ShapeDtypeStruct((_BATCH,), jnp.float32),
        jax.ShapeDtypeStruct((_NW * _L,), jnp.float32),
    ),
    mesh=_mesh,
    compiler_params=pltpu.CompilerParams(
        needs_layout_passes=False, use_tc_tiling_on_sc=True,
        disable_bounds_checks=True),
    scratch_types=(
        pltpu.VMEM((_BPW,), jnp.int32),
        pltpu.VMEM((_BPW,), jnp.int32),
        pltpu.VMEM((_BPW,), jnp.int32),
        pltpu.VMEM((_BPW,), jnp.int32),
        pltpu.VMEM((_BPW,), jnp.int32),
        pltpu.VMEM((_BPW,), jnp.int32),
        pltpu.VMEM((_CH, _WIDE), jnp.float32),
        pltpu.VMEM((_CH, _WIDE), jnp.float32),
        pltpu.VMEM((_CH, _WIDE), jnp.float32),
        pltpu.VMEM((_BPW,), jnp.float32),
        pltpu.VMEM((_L,), jnp.float32),
        pltpu.SemaphoreType.DMA,
    ),
)
def _sc_gather_score(user, pos, neg, uemb, iemb, diff_out, acc_out,
                     uidx, pidx, nidx, urid, prid, nrid,
                     urows, prows, nrows, diffv, accv, sem):
    wid = lax.axis_index("s") * _NC + lax.axis_index("c")
    base = wid * _BPW
    pltpu.sync_copy(user.at[pl.ds(base, _BPW)], uidx)
    pltpu.sync_copy(pos.at[pl.ds(base, _BPW)], pidx)
    pltpu.sync_copy(neg.at[pl.ds(base, _BPW)], nidx)

    def rid_body(k, _):
        s = pl.ds(k * _L, _L)
        urid[s] = lax.shift_right_logical(uidx[s], 3)
        prid[s] = lax.shift_right_logical(pidx[s], 3)
        nrid[s] = lax.shift_right_logical(nidx[s], 3)
        return 0

    lax.fori_loop(0, _BPW // _L, rid_body, 0)

    acc = jnp.zeros((_L,), jnp.float32)
    for c in range(_NCH):
        cs = pl.ds(c * _CH, _CH)
        cp_u = pltpu.async_copy(uemb.at[urid.at[cs]], urows, sem)
        cp_p = pltpu.async_copy(iemb.at[prid.at[cs]], prows, sem)
        cp_n = pltpu.async_copy(iemb.at[nrid.at[cs]], nrows, sem)
        cp_u.wait()
        cp_p.wait()
        cp_n.wait()

        def group_body(g, acc, _c=c):
            rows = g * _L + jnp.arange(_L, dtype=jnp.int32)
            gs = pl.dslice(_c * _CH + g * _L, _L)
            iu = uidx[gs]
            ip = pidx[gs]
            inn = nidx[gs]
            cu = jnp.left_shift(jnp.bitwise_and(iu, _RPW - 1), 4)
            cp = jnp.left_shift(jnp.bitwise_and(ip, _RPW - 1), 4)
            cn = jnp.left_shift(jnp.bitwise_and(inn, _RPW - 1), 4)
            score = jnp.zeros((_L,), jnp.float32)
            for l in range(_EMBED):
                uc = plsc.load_gather(urows, [rows, cu + l])
                pc = plsc.load_gather(prows, [rows, cp + l])
                nc = plsc.load_gather(nrows, [rows, cn + l])
                score = score + uc * (pc - nc)
                acc = acc + uc * uc + pc * pc + nc * nc
            diffv[pl.dslice(_c * _CH + g * _L, _L)] = score
            return acc

        acc = lax.fori_loop(0, _CH // _L, group_body, acc)

    accv[...] = acc
    pltpu.sync_copy(diffv, diff_out.at[pl.ds(base, _BPW)])
    pltpu.sync_copy(accv, acc_out.at[pl.ds(wid * _L, _L)])


def _tc_finish_body(diff_ref, acc_ref, loss_ref, reg_ref):
    d = diff_ref[...]
    ls = jnp.minimum(d, 0.0) - jnp.log1p(jnp.exp(-jnp.abs(d)))
    loss_ref[0, 0] = -jnp.sum(ls) * (1.0 / _BATCH)
    reg_ref[0, 0] = (_REGS * 0.5 / _BATCH) * jnp.sum(acc_ref[...])


def _tc_finish(diff, acc):
    loss, reg = pl.pallas_call(
        _tc_finish_body,
        out_shape=(
            jax.ShapeDtypeStruct((1, 1), jnp.float32),
            jax.ShapeDtypeStruct((1, 1), jnp.float32),
        ),
        out_specs=(
            pl.BlockSpec(memory_space=pltpu.SMEM),
            pl.BlockSpec(memory_space=pltpu.SMEM),
        ),
    )(diff.reshape(_BATCH // 128, 128), acc.reshape(_NW * _L // 128, 128))
    return loss[0, 0], reg[0, 0]


def kernel(user, pos, neg, user_embedding, item_embedding):
    urm, irm = _sc_convert(user_embedding.T, item_embedding.T)
    diff, acc = _sc_gather_score(user, pos, neg, urm, irm)
    loss, reg_loss = _tc_finish(diff, acc)
    return (loss, reg_loss)


# speedup vs baseline: 1.6092x; 1.6032x over previous
import functools

import jax
import jax.numpy as jnp
from jax import lax
from jax.experimental import pallas as pl
from jax.experimental.pallas import tpu as pltpu
from jax.experimental.pallas import tpu_sc as plsc

_EMBED = 16
_BATCH = 16384
_N_ROWS = 1000000
_REGS = 0.0001
_NC, _NS, _L = 2, 16, 16
_NW = _NC * _NS
_BPW = _BATCH // _NW
_CH = 128
_NCH = _BPW // _CH
_WIDE = 128
_RPW = _WIDE // _EMBED
_TCOLS = -(-_N_ROWS // _WIDE)
_OUT_ROWS = _TCOLS * _L

_mesh = plsc.VectorSubcoreMesh(core_axis_name="c", subcore_axis_name="s")


_TPT = 248
_UNIT = 1
_NUNIT = _TPT // _UNIT
_NBUF = 4


@functools.partial(
    pl.kernel,
    out_type=(
        jax.ShapeDtypeStruct((_OUT_ROWS, _WIDE), jnp.float32),
        jax.ShapeDtypeStruct((_OUT_ROWS, _WIDE), jnp.float32),
    ),
    mesh=_mesh,
    compiler_params=pltpu.CompilerParams(
        needs_layout_passes=False, use_tc_tiling_on_sc=True,
        disable_bounds_checks=True),
    scratch_types=(
        pltpu.VMEM((_EMBED, _UNIT * _WIDE), jnp.float32),
        pltpu.VMEM((_EMBED, _UNIT * _WIDE), jnp.float32),
        pltpu.VMEM((_EMBED, _UNIT * _WIDE), jnp.float32),
        pltpu.VMEM((_EMBED, _UNIT * _WIDE), jnp.float32),
        pltpu.VMEM((_UNIT * _EMBED, _WIDE), jnp.float32),
        pltpu.VMEM((_UNIT * _EMBED, _WIDE), jnp.float32),
        pltpu.VMEM((_UNIT * _EMBED, _WIDE), jnp.float32),
        pltpu.VMEM((_UNIT * _EMBED, _WIDE), jnp.float32),
        pltpu.SemaphoreType.DMA,
        pltpu.SemaphoreType.DMA,
        pltpu.SemaphoreType.DMA,
        pltpu.SemaphoreType.DMA,
        pltpu.SemaphoreType.DMA,
        pltpu.SemaphoreType.DMA,
        pltpu.SemaphoreType.DMA,
        pltpu.SemaphoreType.DMA,
        pltpu.VMEM((_EMBED, _L), jnp.int32),
    ),
)
def _sc_convert(uemb_t, iemb_t, urm, irm,
                i0, i1, i2, i3, o0, o1, o2, o3,
                si0, si1, si2, si3, so0, so1, so2, so3, coltab):
    wid = lax.axis_index("s") * _NC + lax.axis_index("c")
    t0 = jnp.minimum(wid * _TPT, _TCOLS - _TPT)
    iota = jnp.arange(_L, dtype=jnp.int32)
    inb = [i0, i1, i2, i3]
    outb = [o0, o1, o2, o3]
    sin = [si0, si1, si2, si3]
    sout = [so0, so1, so2, so3]
    rowv = [2 * m + lax.shift_right_logical(iota, 3) for m in range(_RPW)]
    colbase = jnp.left_shift(jnp.bitwise_and(iota, 7), 4)
    colv = [colbase + l for l in range(_EMBED)]
    del coltab

    def convert_table(src, dst):
        def fetch(k, q):
            col = pl.multiple_of((t0 + k * _UNIT) * _WIDE, _WIDE)
            pltpu.async_copy(
                src.at[:, pl.ds(col, _UNIT * _WIDE)], inb[q], sin[q])

        def wait_fetch(q):
            pltpu.make_async_copy(
                src.at[:, pl.ds(0, _UNIT * _WIDE)], inb[q], sin[q]).wait()

        def flush(k, q):
            row = (t0 + k * _UNIT) * _L
            pltpu.async_copy(
                outb[q], dst.at[pl.ds(row, _UNIT * _L)], sout[q])

        def wait_flush(q):
            pltpu.make_async_copy(
                outb[q], dst.at[pl.ds(0, _UNIT * _L)], sout[q]).wait()

        def process(q):
            for u in range(_UNIT):
                for m in range(_RPW):
                    rv = rowv[m] + u * _L
                    for l in range(_EMBED):
                        v = inb[q][l, pl.ds(u * _WIDE + m * _L, _L)]
                        plsc.store_scatter(outb[q], [rv, colv[l]], v)

        for q in range(_NBUF):
            fetch(jnp.int32(q), q)

        def body(p, _):
            for q in range(_NBUF):
                k = p * _NBUF + q
                wait_fetch(q)

                @pl.when(p >= 1)
                def _(q=q):
                    wait_flush(q)

                process(q)
                flush(k, q)

                @pl.when(k + _NBUF < _NUNIT)
                def _(k=k, q=q):
                    fetch(k + _NBUF, q)

            return 0

        lax.fori_loop(0, _NUNIT // _NBUF, body, 0)
        for q in range(_NBUF):
            wait_flush(q)

    convert_table(uemb_t, urm)
    convert_table(iemb_t, irm)


@functools.partial(
    pl.kernel,
    out_type=(
        jax.ShapeDtypeStruct((_BATCH,), jnp.float32),
        jax.ShapeDtypeStruct((_NW * _L,), jnp.float32),
    ),
    mesh=_mesh,
    compiler_params=pltpu.CompilerParams(
        needs_layout_passes=False, use_tc_tiling_on_sc=True,
        disable_bounds_checks=True),
    scratch_types=(
        pltpu.VMEM((_BPW,), jnp.int32),
        pltpu.VMEM((_BPW,), jnp.int32),
        pltpu.VMEM((_BPW,), jnp.int32),
        pltpu.VMEM((_BPW,), jnp.int32),
        pltpu.VMEM((_BPW,), jnp.int32),
        pltpu.VMEM((_BPW,), jnp.int32),
        pltpu.VMEM((_CH, _WIDE), jnp.float32),
        pltpu.VMEM((_CH, _WIDE), jnp.float32),
        pltpu.VMEM((_CH, _WIDE), jnp.float32),
        pltpu.VMEM((_BPW,), jnp.float32),
        pltpu.VMEM((_L,), jnp.float32),
        pltpu.SemaphoreType.DMA,
    ),
)
def _sc_gather_score(user, pos, neg, uemb, iemb, diff_out, acc_out,
                     uidx, pidx, nidx, urid, prid, nrid,
                     urows, prows, nrows, diffv, accv, sem):
    wid = lax.axis_index("s") * _NC + lax.axis_index("c")
    base = wid * _BPW
    pltpu.sync_copy(user.at[pl.ds(base, _BPW)], uidx)
    pltpu.sync_copy(pos.at[pl.ds(base, _BPW)], pidx)
    pltpu.sync_copy(neg.at[pl.ds(base, _BPW)], nidx)

    def rid_body(k, _):
        s = pl.ds(k * _L, _L)
        urid[s] = lax.shift_right_logical(uidx[s], 3)
        prid[s] = lax.shift_right_logical(pidx[s], 3)
        nrid[s] = lax.shift_right_logical(nidx[s], 3)
        return 0

    lax.fori_loop(0, _BPW // _L, rid_body, 0)

    acc = jnp.zeros((_L,), jnp.float32)
    for c in range(_NCH):
        cs = pl.ds(c * _CH, _CH)
        cp_u = pltpu.async_copy(uemb.at[urid.at[cs]], urows, sem)
        cp_p = pltpu.async_copy(iemb.at[prid.at[cs]], prows, sem)
        cp_n = pltpu.async_copy(iemb.at[nrid.at[cs]], nrows, sem)
        cp_u.wait()
        cp_p.wait()
        cp_n.wait()

        def group_body(g, acc, _c=c):
            rows = g * _L + jnp.arange(_L, dtype=jnp.int32)
            gs = pl.dslice(_c * _CH + g * _L, _L)
            iu = uidx[gs]
            ip = pidx[gs]
            inn = nidx[gs]
            cu = jnp.left_shift(jnp.bitwise_and(iu, _RPW - 1), 4)
            cp = jnp.left_shift(jnp.bitwise_and(ip, _RPW - 1), 4)
            cn = jnp.left_shift(jnp.bitwise_and(inn, _RPW - 1), 4)
            score = jnp.zeros((_L,), jnp.float32)
            for l in range(_EMBED):
                uc = plsc.load_gather(urows, [rows, cu + l])
                pc = plsc.load_gather(prows, [rows, cp + l])
                nc = plsc.load_gather(nrows, [rows, cn + l])
                score = score + uc * (pc - nc)
                acc = acc + uc * uc + pc * pc + nc * nc
            diffv[pl.dslice(_c * _CH + g * _L, _L)] = score
            return acc

        acc = lax.fori_loop(0, _CH // _L, group_body, acc)

    accv[...] = acc
    pltpu.sync_copy(diffv, diff_out.at[pl.ds(base, _BPW)])
    pltpu.sync_copy(accv, acc_out.at[pl.ds(wid * _L, _L)])


def _tc_finish_body(diff_ref, acc_ref, loss_ref, reg_ref):
    d = diff_ref[...]
    ls = jnp.minimum(d, 0.0) - jnp.log1p(jnp.exp(-jnp.abs(d)))
    loss_ref[0, 0] = -jnp.sum(ls) * (1.0 / _BATCH)
    reg_ref[0, 0] = (_REGS * 0.5 / _BATCH) * jnp.sum(acc_ref[...])


def _tc_finish(diff, acc):
    loss, reg = pl.pallas_call(
        _tc_finish_body,
        out_shape=(
            jax.ShapeDtypeStruct((1, 1), jnp.float32),
            jax.ShapeDtypeStruct((1, 1), jnp.float32),
        ),
        out_specs=(
            pl.BlockSpec(memory_space=pltpu.SMEM),
            pl.BlockSpec(memory_space=pltpu.SMEM),
        ),
    )(diff.reshape(_BATCH // 128, 128), acc.reshape(_NW * _L // 128, 128))
    return loss[0, 0], reg[0, 0]


def kernel(user, pos, neg, user_embedding, item_embedding):
    urm, irm = _sc_convert(user_embedding.T, item_embedding.T)
    diff, acc = _sc_gather_score(user, pos, neg, urm, irm)
    loss, reg_loss = _tc_finish(diff, acc)
    return (loss, reg_loss)
